# baseline (speedup 1.0000x reference)
import functools

import jax
import jax.numpy as jnp
from jax import lax
from jax.experimental import pallas as pl
from jax.experimental.pallas import tpu as pltpu
from jax.experimental.pallas import tpu_sc as plsc

N_ROWS = 100000
F = 128
S = 128
L = 16
NC, NS = 2, 16
NW = NC * NS
CHUNK = 160
N_CHUNKS = N_ROWS // CHUNK
GROUPS = CHUNK // L


def _sc_partials(x, batch):
    q, r = divmod(N_CHUNKS, NW)
    mesh = plsc.VectorSubcoreMesh(
        core_axis_name="c", subcore_axis_name="s",
        num_cores=NC, num_subcores=NS)

    @functools.partial(
        pl.kernel,
        mesh=mesh,
        compiler_params=pltpu.CompilerParams(needs_layout_passes=False),
        out_type=[
            jax.ShapeDtypeStruct((NW, S, F), jnp.float32),
            jax.ShapeDtypeStruct((NW, S, F), jnp.float32),
            jax.ShapeDtypeStruct((NW, S, L), jnp.float32),
        ],
        scratch_types=[
            pltpu.VMEM((CHUNK, F), jnp.float32),
            pltpu.VMEM((CHUNK, F), jnp.float32),
            pltpu.VMEM((CHUNK,), jnp.int32),
            pltpu.VMEM((CHUNK,), jnp.int32),
            pltpu.VMEM((S, F), jnp.float32),
            pltpu.VMEM((S, F), jnp.float32),
            pltpu.VMEM((S, L), jnp.float32),
            pltpu.SemaphoreType.DMA,
            pltpu.SemaphoreType.DMA,
            pltpu.SemaphoreType.DMA,
            pltpu.SemaphoreType.DMA,
        ],
    )
    def k(x_hbm, b_hbm, sum_hbm, max_hbm, cnt_hbm,
          xv0, xv1, bv0, bv1, asum, amax, acnt,
          semx0, semx1, semb0, semb1):
        wid = lax.axis_index("s") * NC + lax.axis_index("c")
        lanes = lax.iota(jnp.int32, L)
        zeros = jnp.zeros((L,), jnp.float32)
        ninf = jnp.full((L,), -jnp.inf, jnp.float32)

        c0 = wid * q + jnp.minimum(wid, r)
        c1 = c0 + q + (wid < r).astype(jnp.int32)

        def dma_x(c, xv, semx):
            return pltpu.make_async_copy(
                x_hbm.at[pl.ds(c * CHUNK, CHUNK), :], xv, semx)

        def dma_b(c, bv, semb):
            return pltpu.make_async_copy(
                b_hbm.at[pl.ds(c * CHUNK, CHUNK)], bv, semb)

        def start(c, xv, bv, semx, semb):
            dma_x(c, xv, semx).start()
            dma_b(c, bv, semb).start()

        def wait(c, xv, bv, semx, semb):
            dma_x(c, xv, semx).wait()
            dma_b(c, bv, semb).wait()

        start(c0, xv0, bv0, semx0, semb0)

        def init_body(i2, _):
            acnt[i2, pl.ds(0, L)] = zeros
            return 0

        lax.fori_loop(0, S, init_body, 0)

        def make_row_body(xv, bv):
            def row_body(i, rc):
                prev = rc[0]
                svec = rc[1:9]
                mvec = rc[9:17]
                cnt = rc[17]
                seg = plsc.load_gather(bv, [jnp.full((L,), i, jnp.int32)])
                same = seg == prev
                news, newm = [], []
                for k8 in range(F // L):
                    xk = xv[i, pl.ds(L * k8, L)]
                    sk = jnp.where(same, svec[k8] + xk, xk)
                    mk = jnp.where(same, jnp.maximum(mvec[k8], xk), xk)
                    col = lanes + (L * k8)
                    plsc.store_scatter(asum, [seg, col], sk)
                    plsc.store_scatter(amax, [seg, col], mk)
                    news.append(sk)
                    newm.append(mk)
                newc = jnp.where(same, cnt + 1.0, jnp.ones((L,), jnp.float32))
                plsc.store_scatter(acnt, [seg, lanes], newc)
                return (seg, *news, *newm, newc)
            return row_body

        def process(xv, bv, carry):
            row_body = make_row_body(xv, bv)

            def group_body(g, gc):
                base = g * L
                bvec = bv[pl.ds(base, L)]
                nbase = jnp.minimum(base + L, CHUNK - L)
                bvec_next = bv[pl.ds(nbase, L)]
                fastg = gc[0]
                rest = gc[1:]

                def fast_fn(rc):
                    prev = rc[0]
                    fast_next = jnp.all(bvec_next == prev)
                    svec = rc[1:9]
                    mvec = rc[9:17]
                    cnt = rc[17]
                    news, newm = [], []
                    for k8 in range(F // L):
                        xs16 = [xv[base + j, pl.ds(L * k8, L)]
                                for j in range(L)]
                        if k8 > 0:
                            col = lanes + (L * (k8 - 1))
                            plsc.store_scatter(asum, [bvec, col],
                                               news[k8 - 1])
                            plsc.store_scatter(amax, [bvec, col],
                                               newm[k8 - 1])
                        sacc, macc = svec[k8], mvec[k8]
                        for h in range(2):
                            xs = xs16[8 * h:8 * h + 8]
                            ms = xs
                            while len(xs) > 1:
                                xs = [xs[2 * t] + xs[2 * t + 1]
                                      for t in range(len(xs) // 2)]
                            while len(ms) > 1:
                                ms = [jnp.maximum(ms[2 * t], ms[2 * t + 1])
                                      for t in range(len(ms) // 2)]
                            sacc = sacc + xs[0]
                            macc = jnp.maximum(macc, ms[0])
                        news.append(sacc)
                        newm.append(macc)
                    col = lanes + (L * (F // L - 1))
                    plsc.store_scatter(asum, [bvec, col], news[-1])
                    plsc.store_scatter(amax, [bvec, col], newm[-1])
                    newc = cnt + jnp.float32(L)
                    plsc.store_scatter(acnt, [bvec, lanes], newc)
                    return (fast_next, rc[0], *news, *newm, newc)

                def slow_fn(rc):
                    prev_after = jnp.take_along_axis(
                        bvec, jnp.full((L,), L - 1, jnp.int32), axis=0)
                    fast_next = jnp.all(bvec_next == prev_after)
                    out = lax.fori_loop(base, base + L, row_body, rc)
                    return (fast_next, *out)

                return lax.cond(fastg, fast_fn, slow_fn, rest)

            bvec0 = bv[pl.ds(0, L)]
            fast0 = jnp.all(bvec0 == carry[0])
            out = lax.fori_loop(0, GROUPS, group_body, (fast0, *carry))
            return out[1:]

        def pair_body(p, carry):
            ce = c0 + 2 * p
            co = ce + 1

            @pl.when(co < c1)
            def _():
                start(co, xv1, bv1, semx1, semb1)

            wait(ce, xv0, bv0, semx0, semb0)
            carry = process(xv0, bv0, carry)

            @pl.when(ce + 2 < c1)
            def _():
                start(ce + 2, xv0, bv0, semx0, semb0)

            def odd_fn(rc):
                wait(co, xv1, bv1, semx1, semb1)
                return process(xv1, bv1, rc)

            return lax.cond(co < c1, odd_fn, lambda rc: rc, carry)

        init_carry = (jnp.full((L,), -1, jnp.int32),) \
            + (zeros,) * 8 + (ninf,) * 8 + (zeros,)
        pairs = (c1 - c0 + 1) // 2
        lax.fori_loop(0, pairs, pair_body, init_carry)

        pltpu.sync_copy(asum, sum_hbm.at[wid])
        pltpu.sync_copy(amax, max_hbm.at[wid])
        pltpu.sync_copy(acnt, cnt_hbm.at[wid])

    return k(x, batch)


def _tc_merge(sum_p, max_p, cnt_p):
    NG = 4
    WB = NW // NG

    def body(s_ref, m_ref, c_ref, o_ref, cscr):
        i = pl.program_id(0)
        c = c_ref[...]
        valid = c[:, :, 0:1] > 0.0
        s = jnp.sum(jnp.where(valid, s_ref[...], 0.0), axis=0)
        m = jnp.max(jnp.where(valid, m_ref[...], -jnp.inf), axis=0)
        cn = jnp.sum(c[:, :, 0], axis=0)[:, None]

        @pl.when(i == 0)
        def _():
            o_ref[:, 0:F] = s
            o_ref[:, F:2 * F] = m
            cscr[...] = cn

        @pl.when(i > 0)
        def _():
            o_ref[:, 0:F] += s
            o_ref[:, F:2 * F] = jnp.maximum(o_ref[:, F:2 * F], m)
            cscr[...] += cn

        @pl.when(i == NG - 1)
        def _():
            o_ref[:, 0:F] /= jnp.maximum(cscr[...], 1.0)

    return pl.pallas_call(
        body,
        grid=(NG,),
        in_specs=[
            pl.BlockSpec((WB, S, F), lambda i: (i, 0, 0)),
            pl.BlockSpec((WB, S, F), lambda i: (i, 0, 0)),
            pl.BlockSpec((WB, S, L), lambda i: (i, 0, 0)),
        ],
        out_specs=pl.BlockSpec((S, 2 * F), lambda i: (0, 0)),
        out_shape=jax.ShapeDtypeStruct((S, 2 * F), jnp.float32),
        scratch_shapes=[pltpu.VMEM((S, 1), jnp.float32)],
    )(sum_p, max_p, cnt_p)


def kernel(x, batch):
    sum_p, max_p, cnt_p = _sc_partials(x, batch.astype(jnp.int32))
    return _tc_merge(sum_p, max_p, cnt_p)


# speedup vs baseline: 1.0063x; 1.0063x over previous
import functools

import jax
import jax.numpy as jnp
from jax import lax
from jax.experimental import pallas as pl
from jax.experimental.pallas import tpu as pltpu
from jax.experimental.pallas import tpu_sc as plsc

N_ROWS = 100000
F = 128
S = 128
L = 16
NC, NS = 2, 16
NW = NC * NS
CHUNK = 160
N_CHUNKS = N_ROWS // CHUNK
GROUPS = CHUNK // L


def _sc_partials(x, batch):
    q, r = divmod(N_CHUNKS, NW)
    mesh = plsc.VectorSubcoreMesh(
        core_axis_name="c", subcore_axis_name="s",
        num_cores=NC, num_subcores=NS)

    @functools.partial(
        pl.kernel,
        mesh=mesh,
        compiler_params=pltpu.CompilerParams(needs_layout_passes=False),
        out_type=[
            jax.ShapeDtypeStruct((NW, S, F), jnp.float32),
            jax.ShapeDtypeStruct((NW, S, F), jnp.float32),
            jax.ShapeDtypeStruct((NW, S, L), jnp.float32),
        ],
        scratch_types=[
            pltpu.VMEM((CHUNK, F), jnp.float32),
            pltpu.VMEM((CHUNK, F), jnp.float32),
            pltpu.VMEM((CHUNK,), jnp.int32),
            pltpu.VMEM((CHUNK,), jnp.int32),
            pltpu.VMEM((S, F), jnp.float32),
            pltpu.VMEM((S, F), jnp.float32),
            pltpu.VMEM((S, L), jnp.float32),
            pltpu.SemaphoreType.DMA,
            pltpu.SemaphoreType.DMA,
            pltpu.SemaphoreType.DMA,
            pltpu.SemaphoreType.DMA,
        ],
    )
    def k(x_hbm, b_hbm, sum_hbm, max_hbm, cnt_hbm,
          xv0, xv1, bv0, bv1, asum, amax, acnt,
          semx0, semx1, semb0, semb1):
        wid = lax.axis_index("s") * NC + lax.axis_index("c")
        lanes = lax.iota(jnp.int32, L)
        zeros = jnp.zeros((L,), jnp.float32)
        ninf = jnp.full((L,), -jnp.inf, jnp.float32)

        c0 = wid * q + jnp.minimum(wid, r)
        c1 = c0 + q + (wid < r).astype(jnp.int32)

        def dma_x(c, xv, semx):
            return pltpu.make_async_copy(
                x_hbm.at[pl.ds(c * CHUNK, CHUNK), :], xv, semx)

        def dma_b(c, bv, semb):
            return pltpu.make_async_copy(
                b_hbm.at[pl.ds(c * CHUNK, CHUNK)], bv, semb)

        def start(c, xv, bv, semx, semb):
            dma_x(c, xv, semx).start()
            dma_b(c, bv, semb).start()

        def wait(c, xv, bv, semx, semb):
            dma_x(c, xv, semx).wait()
            dma_b(c, bv, semb).wait()

        start(c0, xv0, bv0, semx0, semb0)

        def init_body(i2, _):
            acnt[i2, pl.ds(0, L)] = zeros
            return 0

        lax.fori_loop(0, S, init_body, 0)

        def make_row_body(xv, bv):
            def row_body(i, rc):
                prev = rc[0]
                svec = rc[1:9]
                mvec = rc[9:17]
                cnt = rc[17]
                seg = plsc.load_gather(bv, [jnp.full((L,), i, jnp.int32)])
                same = seg == prev
                news, newm = [], []
                for k8 in range(F // L):
                    xk = xv[i, pl.ds(L * k8, L)]
                    sk = jnp.where(same, svec[k8] + xk, xk)
                    mk = jnp.where(same, jnp.maximum(mvec[k8], xk), xk)
                    col = lanes + (L * k8)
                    plsc.store_scatter(asum, [seg, col], sk)
                    plsc.store_scatter(amax, [seg, col], mk)
                    news.append(sk)
                    newm.append(mk)
                newc = jnp.where(same, cnt + 1.0, jnp.ones((L,), jnp.float32))
                plsc.store_scatter(acnt, [seg, lanes], newc)
                return (seg, *news, *newm, newc)
            return row_body

        def process(xv, bv, carry):
            row_body = make_row_body(xv, bv)

            def group_body(g, gc):
                base = g * L
                bvec = bv[pl.ds(base, L)]
                nbase = jnp.minimum(base + L, CHUNK - L)
                bvec_next = bv[pl.ds(nbase, L)]
                fastg = gc[0]
                rest = gc[1:]

                def fast_fn(rc):
                    prev = rc[0]
                    fast_next = jnp.all(bvec_next == prev)
                    svec = rc[1:9]
                    mvec = rc[9:17]
                    cnt = rc[17]
                    news, newm = [], []
                    for k8 in range(F // L):
                        xs16 = [xv[base + j, pl.ds(L * k8, L)]
                                for j in range(L)]
                        if k8 > 0:
                            col = lanes + (L * (k8 - 1))
                            plsc.store_scatter(asum, [bvec, col],
                                               news[k8 - 1])
                            plsc.store_scatter(amax, [bvec, col],
                                               newm[k8 - 1])
                        sacc, macc = svec[k8], mvec[k8]
                        for h in range(2):
                            xs = xs16[8 * h:8 * h + 8]
                            ms = xs
                            while len(xs) > 1:
                                xs = [xs[2 * t] + xs[2 * t + 1]
                                      for t in range(len(xs) // 2)]
                            while len(ms) > 1:
                                ms = [jnp.maximum(ms[2 * t], ms[2 * t + 1])
                                      for t in range(len(ms) // 2)]
                            sacc = sacc + xs[0]
                            macc = jnp.maximum(macc, ms[0])
                        news.append(sacc)
                        newm.append(macc)
                    col = lanes + (L * (F // L - 1))
                    plsc.store_scatter(asum, [bvec, col], news[-1])
                    plsc.store_scatter(amax, [bvec, col], newm[-1])
                    newc = cnt + jnp.float32(L)
                    plsc.store_scatter(acnt, [bvec, lanes], newc)
                    return (fast_next, rc[0], *news, *newm, newc)

                def slow_fn(rc):
                    prev_after = jnp.take_along_axis(
                        bvec, jnp.full((L,), L - 1, jnp.int32), axis=0)
                    fast_next = jnp.all(bvec_next == prev_after)
                    out = lax.fori_loop(base, base + L, row_body, rc)
                    return (fast_next, *out)

                return lax.cond(fastg, fast_fn, slow_fn, rest)

            bvec0 = bv[pl.ds(0, L)]
            fast0 = jnp.all(bvec0 == carry[0])
            out = lax.fori_loop(0, GROUPS, group_body, (fast0, *carry))
            return out[1:]

        def pair_body(p, carry):
            ce = c0 + 2 * p
            co = ce + 1

            @pl.when(co < c1)
            def _():
                start(co, xv1, bv1, semx1, semb1)

            wait(ce, xv0, bv0, semx0, semb0)
            carry = process(xv0, bv0, carry)

            @pl.when(ce + 2 < c1)
            def _():
                start(ce + 2, xv0, bv0, semx0, semb0)

            def odd_fn(rc):
                wait(co, xv1, bv1, semx1, semb1)
                return process(xv1, bv1, rc)

            return lax.cond(co < c1, odd_fn, lambda rc: rc, carry)

        init_carry = (jnp.full((L,), -1, jnp.int32),) \
            + (zeros,) * 8 + (ninf,) * 8 + (zeros,)
        pairs = (c1 - c0 + 1) // 2
        lax.fori_loop(0, pairs, pair_body, init_carry)

        pltpu.sync_copy(asum, sum_hbm.at[wid])
        pltpu.sync_copy(amax, max_hbm.at[wid])
        pltpu.sync_copy(acnt, cnt_hbm.at[wid])

    return k(x, batch)


def _tc_merge(sum_p, max_p, cnt_p):
    def body(s_ref, m_ref, c_ref, o_ref):
        c = c_ref[...]
        valid = c[:, :, 0:1] > 0.0
        s = jnp.sum(jnp.where(valid, s_ref[...], 0.0), axis=0)
        m = jnp.max(jnp.where(valid, m_ref[...], -jnp.inf), axis=0)
        cn = jnp.sum(c[:, :, 0], axis=0)[:, None]
        o_ref[:, 0:F] = s / jnp.maximum(cn, 1.0)
        o_ref[:, F:2 * F] = m

    return pl.pallas_call(
        body,
        out_shape=jax.ShapeDtypeStruct((S, 2 * F), jnp.float32),
    )(sum_p, max_p, cnt_p)


def kernel(x, batch):
    sum_p, max_p, cnt_p = _sc_partials(x, batch.astype(jnp.int32))
    return _tc_merge(sum_p, max_p, cnt_p)
